# baseline (speedup 1.0000x reference)
import functools
import math

import jax
import jax.numpy as jnp
from jax import lax
from jax.experimental import pallas as pl
from jax.experimental.pallas import tpu as pltpu
from jax.experimental.pallas import tpu_sc as plsc

NC = 2
NS = 16
NW = NC * NS

D = 64
BATCH = 4096
HIST = 200
LANES = 128
NBUF = 4
VOCAB_ROWS = 1_000_000
SCALE = float(math.sqrt(64.0))


def _sc_body(table_hbm, idx_hbm, out_hbm, idx_v, sbuf, *rest):
    bufs = rest[:NBUF]
    tbufs = rest[NBUF:2 * NBUF]
    gsem = rest[2 * NBUF:3 * NBUF]
    ssem = rest[3 * NBUF:4 * NBUF]
    psem = rest[4 * NBUF]

    blk = lax.axis_index("c") * NS + lax.axis_index("s")

    pcps = [pltpu.async_copy(idx_hbm.at[tt, blk], idx_v.at[tt], psem)
            for tt in range(HIST // 8)]
    for cp in pcps:
        cp.wait()

    @plsc.parallel_loop(0, HIST * LANES // 16, unroll=8)
    def dbl(j):
        idx_v[j >> 6, (j >> 3) & 7, pl.ds((j & 7) * 16, 16)] = (
            idx_v[j >> 6, (j >> 3) & 7, pl.ds((j & 7) * 16, 16)] * 2)

    iota16 = lax.iota(jnp.int32, 16)

    def transpose_scale(buf, tbuf):
        @plsc.parallel_loop(0, LANES, unroll=4)
        def pass1(l):
            rot = (iota16 + l) & 15
            row = jnp.full((16,), 0, jnp.int32) + l
            for c0 in range(0, D, 16):
                v = plsc.load_gather(buf, [row, rot + c0])
                sbuf[l, pl.ds(c0, 16)] = v

        col_idx = [((f % 16 - iota16) & 15) + (f // 16) * 16 for f in range(D)]

        @plsc.parallel_loop(0, LANES // 16, unroll=2)
        def pass2(lb):
            row_idx = lb * 16 + iota16
            for f in range(D):
                v = plsc.load_gather(sbuf, [row_idx, col_idx[f]]) * SCALE
                tbuf[f // 8, 0, f % 8, pl.ds(lb * 16, 16)] = v

    def group(i, carry):
        t0 = i * NBUF
        gcps = []
        for c in range(NBUF):
            t = t0 + c
            gcps.append(pltpu.async_copy(
                table_hbm.at[idx_v.at[t >> 3, t & 7]], bufs[c], gsem[c]))
        scps = []
        for c in range(NBUF):
            t = t0 + c
            gcps[c].wait()
            transpose_scale(bufs[c], tbufs[c])
            scps.append(pltpu.async_copy(
                tbufs[c], out_hbm.at[t, :, pl.ds(blk, 1)], ssem[c]))
        for cp in scps:
            cp.wait()
        return carry

    lax.fori_loop(0, HIST // NBUF, group, 0)


@jax.jit
def kernel(input_ids, embeddings):
    ids_x = (input_ids.astype(jnp.int32).T
             .reshape(HIST // 8, 8, BATCH // LANES, LANES)
             .transpose(0, 2, 1, 3))
    table2 = jnp.pad(embeddings, ((0, 0), (0, D))).reshape(2 * VOCAB_ROWS, D)
    mesh = plsc.VectorSubcoreMesh(core_axis_name="c", subcore_axis_name="s")
    k = functools.partial(
        pl.kernel,
        mesh=mesh,
        out_type=jax.ShapeDtypeStruct(
            (HIST, D // 8, BATCH // LANES, 8, LANES), jnp.float32),
        scratch_types=(
            [pltpu.VMEM((HIST // 8, 8, LANES), jnp.int32)]
            + [pltpu.VMEM((LANES, D), jnp.float32)]
            + [pltpu.VMEM((LANES, D), jnp.float32) for _ in range(NBUF)]
            + [pltpu.VMEM((D // 8, 1, 8, LANES), jnp.float32)
               for _ in range(NBUF)]
            + [pltpu.SemaphoreType.DMA for _ in range(2 * NBUF + 1)]
        ),
        compiler_params=pltpu.CompilerParams(
            use_tc_tiling_on_sc=False, needs_layout_passes=False),
    )(_sc_body)
    y = k(table2, ids_x)
    return y.transpose(2, 4, 0, 1, 3).reshape(BATCH, HIST, D)


# speedup vs baseline: 1.0366x; 1.0366x over previous
import functools
import math

import jax
import jax.numpy as jnp
from jax import lax
from jax.experimental import pallas as pl
from jax.experimental.pallas import tpu as pltpu
from jax.experimental.pallas import tpu_sc as plsc

NC = 2
NS = 16
NW = NC * NS

VOCAB = 1_000_000
D = 64
N_IDS = 4096 * 200
PER_W = N_IDS // NW
CHUNK = 128
NCHUNK = PER_W // CHUNK
NBUF = 8
SCALE = float(math.sqrt(64.0))


def _sc_body(table_hbm, idx_hbm, out_hbm, idx_v, *bufs_and_sems):
    rows = bufs_and_sems[:NBUF]
    gsem = bufs_and_sems[NBUF:2 * NBUF]
    ssem = bufs_and_sems[2 * NBUF:3 * NBUF]

    wid = lax.axis_index("c") * NS + lax.axis_index("s")

    pltpu.sync_copy(idx_hbm.at[wid], idx_v)

    def scale_rows(buf):
        def sbody(r4, carry):
            for rr in range(4):
                row = r4 * 4 + rr
                for c in range(0, D, 16):
                    buf[row, pl.ds(c, 16)] = buf[row, pl.ds(c, 16)] * SCALE
            return carry
        lax.fori_loop(0, CHUNK // 4, sbody, 0)

    def outer(i, carry):
        g0 = i * NBUF
        gcps = []
        for b in range(NBUF):
            gcps.append(pltpu.async_copy(
                table_hbm.at[idx_v.at[g0 + b]], rows[b], gsem[b]))
        scps = []
        for b in range(NBUF):
            gcps[b].wait()
            scale_rows(rows[b])
            scps.append(pltpu.async_copy(
                rows[b], out_hbm.at[wid, g0 + b], ssem[b]))
        for b in range(NBUF):
            scps[b].wait()
        return carry

    lax.fori_loop(0, NCHUNK // NBUF, outer, 0)


@jax.jit
def kernel(input_ids, embeddings):
    idx = input_ids.reshape(NW, NCHUNK, CHUNK).astype(jnp.int32)
    mesh = plsc.VectorSubcoreMesh(core_axis_name="c", subcore_axis_name="s")
    k = functools.partial(
        pl.kernel,
        mesh=mesh,
        out_type=jax.ShapeDtypeStruct((NW, NCHUNK, CHUNK, D), jnp.float32),
        scratch_types=(
            [pltpu.VMEM((NCHUNK, CHUNK), jnp.int32)]
            + [pltpu.VMEM((CHUNK, D), jnp.float32) for _ in range(NBUF)]
            + [pltpu.SemaphoreType.DMA for _ in range(2 * NBUF)]
        ),
        compiler_params=pltpu.CompilerParams(use_tc_tiling_on_sc=False),
    )(_sc_body)
    out = k(embeddings, idx)
    return out.reshape(4096, 200, D)
